# Initial kernel scaffold; baseline (speedup 1.0000x reference)
import jax
import jax.numpy as jnp
from jax.experimental import pallas as pl


def kernel(pred, target, label_weight):
    raise NotImplementedError("write your pallas kernel here")


# speedup vs baseline: 1.9891x; 1.9891x over previous
import functools

import jax
import jax.numpy as jnp
from jax import lax
from jax.experimental import pallas as pl
from jax.experimental.pallas import tpu as pltpu
from jax.experimental.pallas import tpu_sc as plsc

_MU = 0.02
_BINS = 10
_LOSS_WEIGHT = 1.0

_L = 16
_NC = 2
_NS = 16
_NW = _NC * _NS
_BINS_PAD = 16
_CHUNK = 8192


def _make_hist_body(q, chunk):
    nfull = q // chunk
    tail = q - nfull * chunk
    tail_vecs = tail // _L
    tail_rem = tail - tail_vecs * _L
    assert tail % 8 == 0

    def body(pred_hbm, target_hbm, lw_hbm, cnt_out, sum_out,
             bufs, hcnt, hsum, sems):
        wid = lax.axis_index("s") * _NC + lax.axis_index("c")
        base = wid * q

        lane = lax.iota(jnp.int32, _L)
        zeros = jnp.zeros((_L,), jnp.float32)
        for r in range(_BINS_PAD):
            hcnt[pl.ds(r * _L, _L)] = zeros
            hsum[pl.ds(r * _L, _L)] = zeros

        srcs = (pred_hbm, target_hbm, lw_hbm)
        sizes = [chunk] * nfull + ([tail] if tail else [])
        nchunks = len(sizes)

        def start(j, slot):
            off = base + j * chunk
            sz = sizes[j]
            return [
                pltpu.make_async_copy(
                    srcs[a].at[pl.ds(off, sz)],
                    bufs[slot][a].at[pl.ds(0, sz)],
                    sems[slot][a])
                for a in range(3)
            ]

        copies = {0: start(0, 0)}
        for c in copies[0]:
            c.start()

        mu2 = jnp.float32(_MU * _MU)

        def make_step(bp, bt, bw, mask):
            def step(k, carry):
                off = k * _L
                p = bp[pl.ds(off, _L)]
                t = bt[pl.ds(off, _L)]
                w = bw[pl.ds(off, _L)]
                d = p - t
                x = d * d + mu2
                xi = lax.bitcast_convert_type(x, jnp.int32)
                yi = jnp.int32(0x5F3759DF) - (xi >> 1)
                y = lax.bitcast_convert_type(yi, jnp.float32)
                y = y * (1.5 - 0.5 * x * y * y)
                y = y * (1.5 - 0.5 * x * y * y)
                loss = x * y - _MU
                g10 = jnp.abs(d) * y * 10.0
                bini = jnp.minimum(g10.astype(jnp.int32), _BINS - 1)
                slot_idx = bini * _L + lane
                validm = w > 0.0
                cntv = jnp.where(validm, 1.0, 0.0).astype(jnp.float32)
                lossv = jnp.where(validm, loss, 0.0)
                plsc.addupdate_scatter(hcnt, [slot_idx], cntv, mask=mask)
                plsc.addupdate_scatter(hsum, [slot_idx], lossv, mask=mask)
                return carry

            return step

        for j in range(nchunks):
            slot = j % 2
            if j + 1 < nchunks:
                copies[j + 1] = start(j + 1, 1 - slot)
                for c in copies[j + 1]:
                    c.start()
            for c in copies.pop(j):
                c.wait()
            bp, bt, bw = bufs[slot]
            full_step = make_step(bp, bt, bw, None)
            if sizes[j] == chunk:
                lax.fori_loop(0, chunk // _L, full_step, 0, unroll=4)
            else:
                if tail_vecs:
                    lax.fori_loop(0, tail_vecs, full_step, 0, unroll=4)
                if tail_rem:
                    tail_step = make_step(bp, bt, bw, lane < tail_rem)
                    tail_step(jnp.int32(tail_vecs), 0)

        pltpu.sync_copy(hcnt, cnt_out.at[wid])
        pltpu.sync_copy(hsum, sum_out.at[wid])

    return body


def _epilogue_body(cnt_ref, sum_ref, out_ref):
    c = cnt_ref[...]
    s = sum_ref[...]
    cb = jnp.sum(jnp.sum(c, axis=0), axis=1, keepdims=True)
    sb = jnp.sum(jnp.sum(s, axis=0), axis=1, keepdims=True)
    nz = cb > 0.0
    n = jnp.sum(nz.astype(jnp.float32))
    denom = jnp.where(nz, cb * n, 1.0)
    contrib = jnp.where(nz, sb / denom, 0.0)
    total = jnp.sum(contrib, keepdims=True) * jnp.float32(_LOSS_WEIGHT)
    out_ref[...] = total.reshape(1, 1)


def kernel(pred, target, label_weight):
    total = pred.size
    assert total % _NW == 0
    q = total // _NW

    p = pred.reshape(-1)
    t = target.reshape(-1)
    w = label_weight.reshape(-1)

    mesh = plsc.VectorSubcoreMesh(core_axis_name="c", subcore_axis_name="s")
    hist = pl.kernel(
        _make_hist_body(q, _CHUNK),
        out_type=(
            jax.ShapeDtypeStruct((_NW, _BINS_PAD * _L), jnp.float32),
            jax.ShapeDtypeStruct((_NW, _BINS_PAD * _L), jnp.float32),
        ),
        mesh=mesh,
        scratch_types=(
            tuple(tuple(pltpu.VMEM((_CHUNK,), jnp.float32) for _ in range(3))
                  for _ in range(2)),
            pltpu.VMEM((_BINS_PAD * _L,), jnp.float32),
            pltpu.VMEM((_BINS_PAD * _L,), jnp.float32),
            tuple(tuple(pltpu.SemaphoreType.DMA for _ in range(3))
                  for _ in range(2)),
        ),
        compiler_params=pltpu.CompilerParams(needs_layout_passes=False),
    )
    cnt, sums = hist(p, t, w)
    cnt = cnt.reshape(_NW, _BINS_PAD, _L)
    sums = sums.reshape(_NW, _BINS_PAD, _L)

    out = pl.pallas_call(
        _epilogue_body,
        out_shape=jax.ShapeDtypeStruct((1, 1), jnp.float32),
    )(cnt, sums)
    return out[0, 0]
